# baseline (speedup 1.0000x reference)
import functools

import jax
import jax.numpy as jnp
from jax import lax
from jax.experimental import pallas as pl
from jax.experimental.pallas import tpu as pltpu
from jax.experimental.pallas import tpu_sc as plsc

N = 65536
D_IN = 4
H = 512
Z = 128
K = 1024
BETA = 0.25
NU = 0.1

BLOCK_ROWS = 4096
CHAIN_ROWS = 1024

_INV_SQRT2 = 0.7071067811865476


def _gelu(x):
    return 0.5 * x * (1.0 + jax.lax.erf(x * _INV_SQRT2))


def _encode_body(x_ref, mean_ref, std_ref,
                 w1_ref, b1_ref, w2_ref, b2_ref, w3_ref, b3_ref,
                 cb_ref, idx_ref, loss_ref):
    i = pl.program_id(0)

    cb = cb_ref[...]
    cb_sq = jnp.sum(cb * cb, axis=1)[None, :]
    mean = mean_ref[...]
    std = std_ref[...]

    sqs = []
    for s in range(BLOCK_ROWS // CHAIN_ROWS):
        xn = (x_ref[pl.ds(s * CHAIN_ROWS, CHAIN_ROWS), :] - mean) / std

        h = _gelu(xn @ w1_ref[...] + b1_ref[...])
        h = _gelu(h @ w2_ref[...] + b2_ref[...])
        z = h @ w3_ref[...] + b3_ref[...]

        z_sq = jnp.sum(z * z, axis=1, keepdims=True)
        cross = jax.lax.dot_general(z, cb, (((1,), (1,)), ((), ())))
        d = z_sq - 2.0 * cross + cb_sq

        dmin = jnp.min(d, axis=1, keepdims=True)
        iota = jax.lax.broadcasted_iota(jnp.int32, d.shape, 1)
        idx = jnp.min(jnp.where(d == dmin, iota, K), axis=1)
        idx_ref[pl.ds(s * CHAIN_ROWS, CHAIN_ROWS), :] = idx[:, None]

        sqs.append(jnp.sum(dmin))

    sq = sum(sqs[1:], sqs[0]).reshape(1, 1)

    @pl.when(i == 0)
    def _init():
        loss_ref[...] = sq

    @pl.when(i > 0)
    def _acc():
        loss_ref[...] += sq


def _decode_body(zq_ref, mean_ref, std_ref,
                 dw1_ref, db1_ref, dw2_ref, db2_ref, dw3_ref, db3_ref,
                 out_ref):
    mean = mean_ref[...]
    std = std_ref[...]
    for s in range(BLOCK_ROWS // CHAIN_ROWS):
        z_q = zq_ref[pl.ds(s * CHAIN_ROWS, CHAIN_ROWS), :]
        g = _gelu(z_q @ dw1_ref[...] + db1_ref[...])
        g = _gelu(g @ dw2_ref[...] + db2_ref[...])
        xr = g @ dw3_ref[...] + db3_ref[...]
        out_ref[pl.ds(s * CHAIN_ROWS, CHAIN_ROWS), :] = xr * std + mean


_SC_INFO = plsc.get_sparse_core_info()
_NW = _SC_INFO.num_cores * _SC_INFO.num_subcores
_B_PER_W = N // _NW
_CHUNK = 128
_NCHUNK = _B_PER_W // _CHUNK


@functools.partial(
    pl.kernel,
    mesh=plsc.VectorSubcoreMesh(core_axis_name="c", subcore_axis_name="s"),
    out_type=jax.ShapeDtypeStruct((N, Z), jnp.float32),
    scratch_types=[
        pltpu.VMEM((_CHUNK,), jnp.int32),
        pltpu.VMEM((_CHUNK, Z), jnp.float32),
        pltpu.SemaphoreType.DMA,
    ],
)
def _sc_gather(table_hbm, idx_hbm, out_hbm, idx_v, rows_v, sem):
    wid = lax.axis_index("s") * _SC_INFO.num_cores + lax.axis_index("c")
    base = wid * _B_PER_W

    def body(c, carry):
        off = base + c * _CHUNK
        pltpu.sync_copy(idx_hbm.at[pl.ds(off, _CHUNK)], idx_v)
        pltpu.async_copy(table_hbm.at[idx_v], rows_v, sem).wait()
        pltpu.sync_copy(rows_v, out_hbm.at[pl.ds(off, _CHUNK)])
        return carry

    lax.fori_loop(0, _NCHUNK, body, 0)


@jax.jit
def kernel(x, mean, std, enc_w1, enc_b1, enc_w2, enc_b2, enc_w3, enc_b3,
           codebook, affine_scale, affine_bias,
           dec_w1, dec_b1, dec_w2, dec_b2, dec_w3, dec_b3):
    n = x.shape[0]
    grid = (n // BLOCK_ROWS,)

    cb_affine = codebook * affine_scale + affine_bias

    b1 = enc_b1.reshape(1, H)
    b2 = enc_b2.reshape(1, H)
    b3 = enc_b3.reshape(1, Z)
    db1 = dec_b1.reshape(1, H)
    db2 = dec_b2.reshape(1, H)
    db3 = dec_b3.reshape(1, D_IN)

    def fixed(shape):
        return pl.BlockSpec(shape, lambda i: (0,) * len(shape))

    idx2d, loss_sum = pl.pallas_call(
        _encode_body,
        grid=grid,
        in_specs=[
            pl.BlockSpec((BLOCK_ROWS, D_IN), lambda i: (i, 0)),
            fixed((1, D_IN)), fixed((1, D_IN)),
            fixed((D_IN, H)), fixed((1, H)),
            fixed((H, H)), fixed((1, H)),
            fixed((H, Z)), fixed((1, Z)),
            fixed((K, Z)),
        ],
        out_specs=[
            pl.BlockSpec((BLOCK_ROWS, 1), lambda i: (i, 0)),
            pl.BlockSpec((1, 1), lambda i: (0, 0)),
        ],
        out_shape=[
            jax.ShapeDtypeStruct((n, 1), jnp.int32),
            jax.ShapeDtypeStruct((1, 1), jnp.float32),
        ],
    )(x, mean, std, enc_w1, b1, enc_w2, b2, enc_w3, b3, cb_affine)

    z_q = _sc_gather(cb_affine, idx2d.reshape(n))

    out = pl.pallas_call(
        _decode_body,
        grid=grid,
        in_specs=[
            pl.BlockSpec((BLOCK_ROWS, Z), lambda i: (i, 0)),
            fixed((1, D_IN)), fixed((1, D_IN)),
            fixed((Z, H)), fixed((1, H)),
            fixed((H, H)), fixed((1, H)),
            fixed((H, D_IN)), fixed((1, D_IN)),
        ],
        out_specs=pl.BlockSpec((BLOCK_ROWS, D_IN), lambda i: (i, 0)),
        out_shape=jax.ShapeDtypeStruct((n, D_IN), jnp.float32),
    )(z_q, mean, std, dec_w1, db1, dec_w2, db2, dec_w3, db3)

    m = loss_sum[0, 0] / (n * Z)
    vq_loss = (1.0 - BETA) * m + BETA * m
    return (out, vq_loss)


# speedup vs baseline: 1.0012x; 1.0012x over previous
import functools

import jax
import jax.numpy as jnp
from jax import lax
from jax.experimental import pallas as pl
from jax.experimental.pallas import tpu as pltpu
from jax.experimental.pallas import tpu_sc as plsc

N = 65536
D_IN = 4
H = 512
Z = 128
K = 1024
BETA = 0.25
NU = 0.1

BLOCK_ROWS = 4096
CHAIN_ROWS = 1024

_INV_SQRT2 = 0.7071067811865476


def _gelu(x):
    return 0.5 * x * (1.0 + jax.lax.erf(x * _INV_SQRT2))


def _encode_body(x_ref, mean_ref, std_ref,
                 w1_ref, b1_ref, w2_ref, b2_ref, w3_ref, b3_ref,
                 cb_ref, idx_ref, loss_ref):
    i = pl.program_id(0)

    cb = cb_ref[...]
    cb_sq = jnp.sum(cb * cb, axis=1)[None, :]
    mean = mean_ref[...]
    std = std_ref[...]

    sqs = []
    for s in range(BLOCK_ROWS // CHAIN_ROWS):
        xn = (x_ref[pl.ds(s * CHAIN_ROWS, CHAIN_ROWS), :] - mean) / std

        h = _gelu(xn @ w1_ref[...] + b1_ref[...])
        h = _gelu(h @ w2_ref[...] + b2_ref[...])
        z = h @ w3_ref[...] + b3_ref[...]

        z_sq = jnp.sum(z * z, axis=1, keepdims=True)
        cross = jax.lax.dot_general(z, cb, (((1,), (1,)), ((), ())))
        d = z_sq - 2.0 * cross + cb_sq

        dmin = jnp.min(d, axis=1, keepdims=True)
        iota = jax.lax.broadcasted_iota(jnp.int32, d.shape, 1)
        idx = jnp.min(jnp.where(d == dmin, iota, K), axis=1)
        idx_ref[pl.ds(s * CHAIN_ROWS, CHAIN_ROWS), :] = idx[:, None]

        sqs.append(jnp.sum(dmin))

    sq = sum(sqs[1:], sqs[0]).reshape(1, 1)

    @pl.when(i == 0)
    def _init():
        loss_ref[...] = sq

    @pl.when(i > 0)
    def _acc():
        loss_ref[...] += sq


def _decode_body(zq_ref, mean_ref, std_ref,
                 dw1_ref, db1_ref, dw2_ref, db2_ref, dw3_ref, db3_ref,
                 out_ref):
    mean = mean_ref[...]
    std = std_ref[...]
    for s in range(BLOCK_ROWS // CHAIN_ROWS):
        z_q = zq_ref[pl.ds(s * CHAIN_ROWS, CHAIN_ROWS), :]
        g = _gelu(z_q @ dw1_ref[...] + db1_ref[...])
        g = _gelu(g @ dw2_ref[...] + db2_ref[...])
        xr = g @ dw3_ref[...] + db3_ref[...]
        out_ref[pl.ds(s * CHAIN_ROWS, CHAIN_ROWS), :] = xr * std + mean


_SC_INFO = plsc.get_sparse_core_info()
_NW = _SC_INFO.num_cores * _SC_INFO.num_subcores
_B_PER_W = N // _NW
_CHUNK = 512
_NCHUNK = _B_PER_W // _CHUNK


@functools.partial(
    pl.kernel,
    mesh=plsc.VectorSubcoreMesh(core_axis_name="c", subcore_axis_name="s"),
    out_type=jax.ShapeDtypeStruct((N, Z), jnp.float32),
    scratch_types=[
        pltpu.VMEM((_CHUNK,), jnp.int32),
        pltpu.VMEM((_CHUNK, Z), jnp.float32),
        pltpu.SemaphoreType.DMA,
    ],
)
def _sc_gather(table_hbm, idx_hbm, out_hbm, idx_v, rows_v, sem):
    wid = lax.axis_index("s") * _SC_INFO.num_cores + lax.axis_index("c")
    base = wid * _B_PER_W

    for c in range(_NCHUNK):
        off = base + c * _CHUNK
        pltpu.sync_copy(idx_hbm.at[pl.ds(off, _CHUNK)], idx_v)
        pltpu.async_copy(table_hbm.at[idx_v], rows_v, sem).wait()
        pltpu.sync_copy(rows_v, out_hbm.at[pl.ds(off, _CHUNK)])


@jax.jit
def kernel(x, mean, std, enc_w1, enc_b1, enc_w2, enc_b2, enc_w3, enc_b3,
           codebook, affine_scale, affine_bias,
           dec_w1, dec_b1, dec_w2, dec_b2, dec_w3, dec_b3):
    n = x.shape[0]
    grid = (n // BLOCK_ROWS,)

    cb_affine = codebook * affine_scale + affine_bias

    b1 = enc_b1.reshape(1, H)
    b2 = enc_b2.reshape(1, H)
    b3 = enc_b3.reshape(1, Z)
    db1 = dec_b1.reshape(1, H)
    db2 = dec_b2.reshape(1, H)
    db3 = dec_b3.reshape(1, D_IN)

    def fixed(shape):
        return pl.BlockSpec(shape, lambda i: (0,) * len(shape))

    idx2d, loss_sum = pl.pallas_call(
        _encode_body,
        grid=grid,
        in_specs=[
            pl.BlockSpec((BLOCK_ROWS, D_IN), lambda i: (i, 0)),
            fixed((1, D_IN)), fixed((1, D_IN)),
            fixed((D_IN, H)), fixed((1, H)),
            fixed((H, H)), fixed((1, H)),
            fixed((H, Z)), fixed((1, Z)),
            fixed((K, Z)),
        ],
        out_specs=[
            pl.BlockSpec((BLOCK_ROWS, 1), lambda i: (i, 0)),
            pl.BlockSpec((1, 1), lambda i: (0, 0)),
        ],
        out_shape=[
            jax.ShapeDtypeStruct((n, 1), jnp.int32),
            jax.ShapeDtypeStruct((1, 1), jnp.float32),
        ],
    )(x, mean, std, enc_w1, b1, enc_w2, b2, enc_w3, b3, cb_affine)

    z_q = _sc_gather(cb_affine, idx2d.reshape(n))

    out = pl.pallas_call(
        _decode_body,
        grid=grid,
        in_specs=[
            pl.BlockSpec((BLOCK_ROWS, Z), lambda i: (i, 0)),
            fixed((1, D_IN)), fixed((1, D_IN)),
            fixed((Z, H)), fixed((1, H)),
            fixed((H, H)), fixed((1, H)),
            fixed((H, D_IN)), fixed((1, D_IN)),
        ],
        out_specs=pl.BlockSpec((BLOCK_ROWS, D_IN), lambda i: (i, 0)),
        out_shape=jax.ShapeDtypeStruct((n, D_IN), jnp.float32),
    )(z_q, mean, std, dec_w1, db1, dec_w2, db2, dec_w3, db3)

    m = loss_sum[0, 0] / (n * Z)
    vq_loss = (1.0 - BETA) * m + BETA * m
    return (out, vq_loss)


# speedup vs baseline: 3.1263x; 3.1225x over previous
import jax
import jax.numpy as jnp
from jax.experimental import pallas as pl

N = 65536
D_IN = 4
H = 512
Z = 128
K = 1024
BETA = 0.25
NU = 0.1

BLOCK_ROWS = 4096
CHAIN_ROWS = 1024


_INV_SQRT2 = 0.7071067811865476


def _gelu(x):
    return 0.5 * x * (1.0 + jax.lax.erf(x * _INV_SQRT2))


def _vqvae_body(x_ref, mean_ref, std_ref,
                w1_ref, b1_ref, w2_ref, b2_ref, w3_ref, b3_ref,
                cb_ref, asc_ref, abi_ref,
                dw1_ref, db1_ref, dw2_ref, db2_ref, dw3_ref, db3_ref,
                out_ref, loss_ref):
    i = pl.program_id(0)

    cb = cb_ref[...] * asc_ref[...] + abi_ref[...]
    cb_sq = jnp.sum(cb * cb, axis=1)[None, :]
    cb_hi = cb.astype(jnp.bfloat16)
    cb_lo = (cb - cb_hi.astype(jnp.float32)).astype(jnp.bfloat16)

    mean = mean_ref[...]
    std = std_ref[...]

    half = CHAIN_ROWS
    sqs = []
    for s in range(BLOCK_ROWS // CHAIN_ROWS):
        xn = (x_ref[pl.ds(s * half, half), :] - mean) / std

        h = _gelu(xn @ w1_ref[...] + b1_ref[...])
        h = _gelu(h @ w2_ref[...] + b2_ref[...])
        z = h @ w3_ref[...] + b3_ref[...]

        z_sq = jnp.sum(z * z, axis=1, keepdims=True)
        cross = jax.lax.dot_general(z, cb, (((1,), (1,)), ((), ())))
        d = z_sq - 2.0 * cross + cb_sq

        dmin = jnp.min(d, axis=1, keepdims=True)
        iota = jax.lax.broadcasted_iota(jnp.int32, d.shape, 1)
        idx = jnp.min(jnp.where(d == dmin, iota, K), axis=1)
        onehot = (iota == idx[:, None]).astype(jnp.bfloat16)
        dn = (((1,), (0,)), ((), ()))
        z_q = (jax.lax.dot_general(onehot, cb_hi, dn,
                                   preferred_element_type=jnp.float32)
               + jax.lax.dot_general(onehot, cb_lo, dn,
                                     preferred_element_type=jnp.float32))

        diff = z - z_q
        sqs.append(jnp.sum(diff * diff))

        z_q_st = z + (z_q - z)
        g = _gelu(z_q_st @ dw1_ref[...] + db1_ref[...])
        g = _gelu(g @ dw2_ref[...] + db2_ref[...])
        xr = g @ dw3_ref[...] + db3_ref[...]
        out_ref[pl.ds(s * half, half), :] = xr * std + mean

    sq = sum(sqs[1:], sqs[0]).reshape(1, 1)

    @pl.when(i == 0)
    def _init():
        loss_ref[...] = sq

    @pl.when(i > 0)
    def _acc():
        loss_ref[...] += sq


@jax.jit
def kernel(x, mean, std, enc_w1, enc_b1, enc_w2, enc_b2, enc_w3, enc_b3,
           codebook, affine_scale, affine_bias,
           dec_w1, dec_b1, dec_w2, dec_b2, dec_w3, dec_b3):
    n = x.shape[0]
    grid = (n // BLOCK_ROWS,)

    b1 = enc_b1.reshape(1, H)
    b2 = enc_b2.reshape(1, H)
    b3 = enc_b3.reshape(1, Z)
    db1 = dec_b1.reshape(1, H)
    db2 = dec_b2.reshape(1, H)
    db3 = dec_b3.reshape(1, D_IN)

    def fixed(shape):
        return pl.BlockSpec(shape, lambda i: (0,) * len(shape))

    out, loss_sum = pl.pallas_call(
        _vqvae_body,
        grid=grid,
        in_specs=[
            pl.BlockSpec((BLOCK_ROWS, D_IN), lambda i: (i, 0)),
            fixed((1, D_IN)), fixed((1, D_IN)),
            fixed((D_IN, H)), fixed((1, H)),
            fixed((H, H)), fixed((1, H)),
            fixed((H, Z)), fixed((1, Z)),
            fixed((K, Z)), fixed((1, Z)), fixed((1, Z)),
            fixed((Z, H)), fixed((1, H)),
            fixed((H, H)), fixed((1, H)),
            fixed((H, D_IN)), fixed((1, D_IN)),
        ],
        out_specs=[
            pl.BlockSpec((BLOCK_ROWS, D_IN), lambda i: (i, 0)),
            pl.BlockSpec((1, 1), lambda i: (0, 0)),
        ],
        out_shape=[
            jax.ShapeDtypeStruct((n, D_IN), jnp.float32),
            jax.ShapeDtypeStruct((1, 1), jnp.float32),
        ],
    )(x, mean, std, enc_w1, b1, enc_w2, b2, enc_w3, b3,
      codebook, affine_scale, affine_bias,
      dec_w1, db1, dec_w2, db2, dec_w3, db3)

    m = loss_sum[0, 0] / (n * Z)
    vq_loss = (1.0 - BETA) * m + BETA * m
    return (out, vq_loss)
